# Initial kernel scaffold; baseline (speedup 1.0000x reference)
import jax
import jax.numpy as jnp
from jax.experimental import pallas as pl


def kernel(cf_0, cf_1, cf_2, cf_3, cf_4, cf_5, cf_6, cf_7, df_0, df_1, df_2, df_3, df_4, df_5, df_6, df_7, df_8, df_9, df_10, df_11, df_12, df_13, df_14, df_15, df_16, df_17, df_18, df_19, df_20, df_21, df_22, df_23, df_24, df_25, W1, b1, W2, b2, tables):
    raise NotImplementedError("write your pallas kernel here")


# speedup vs baseline: 1.0612x; 1.0612x over previous
import functools

import jax
import jax.numpy as jnp
from jax import lax
from jax.experimental import pallas as pl
from jax.experimental.pallas import tpu as pltpu
from jax.experimental.pallas import tpu_sc as plsc

_NUM_CF = 8
_NUM_DF = 26
_VOCAB = 100000
_EMBED = 64
_BATCH = 16384

_INFO = plsc.get_sparse_core_info()
_NC = _INFO.num_cores
_NS = _INFO.num_subcores
_NW = _NC * _NS
_BPW = _BATCH // _NW
_IDXW = 128
_NG = _BPW // _IDXW


def _sc_embsum(didx2d, tables_flat):
    mesh = plsc.VectorSubcoreMesh(core_axis_name="c", subcore_axis_name="s")

    @functools.partial(
        pl.kernel,
        out_type=jax.ShapeDtypeStruct((_BATCH, _EMBED), jnp.float32),
        mesh=mesh,
        scratch_types=[
            pltpu.VMEM((_NG, _IDXW), jnp.int32),
            pltpu.VMEM((_NG, _IDXW), jnp.int32),
            pltpu.VMEM((_BPW, _EMBED), jnp.float32),
            pltpu.VMEM((_BPW, _EMBED), jnp.float32),
            pltpu.VMEM((_BPW, _EMBED), jnp.float32),
            pltpu.SemaphoreType.DMA,
            pltpu.SemaphoreType.DMA,
        ],
        compiler_params=pltpu.CompilerParams(use_tc_tiling_on_sc=False),
    )
    def body(didx_hbm, tab_hbm, out_hbm, idx0, idx1, rows0, rows1, acc, sem0, sem1):
        wid = lax.axis_index("s") * _NC + lax.axis_index("c")
        idx_bufs = (idx0, idx1)
        rows_bufs = (rows0, rows1)
        sems = (sem0, sem1)
        rows_per_feat = _BATCH // _IDXW

        def stage_idx(i, slot):
            ib = idx_bufs[slot]
            pltpu.sync_copy(
                didx_hbm.at[pl.ds(i * rows_per_feat + wid * _NG, _NG)], ib)
            off = i * _VOCAB

            def add_off(j, _):
                g = j // (_IDXW // 16)
                c = j % (_IDXW // 16)
                ib[g, pl.ds(c * 16, 16)] = ib[g, pl.ds(c * 16, 16)] + off
                return 0

            lax.fori_loop(0, _NG * (_IDXW // 16), add_off, 0, unroll=4)

        def fire_gathers(slot, dst):
            ib = idx_bufs[slot]
            handles = []
            for g in range(_NG):
                handles.append(
                    pltpu.async_copy(
                        tab_hbm.at[ib.at[g]],
                        dst.at[pl.ds(g * _IDXW, _IDXW)],
                        sems[slot],
                    ))
            return handles

        def accumulate(slot):
            rb = rows_bufs[slot]

            def accrow(r, _):
                for c in range(_EMBED // 16):
                    plsc.addupdate(
                        acc.at[r, pl.ds(c * 16, 16)],
                        rb[r, pl.ds(c * 16, 16)])
                return 0

            lax.fori_loop(0, _BPW, accrow, 0, unroll=4)

        stage_idx(0, 0)
        h0 = fire_gathers(0, acc)
        stage_idx(1, 1)
        h1 = fire_gathers(1, rows_bufs[1])
        for h in h0:
            h.wait()
        pending = h1
        for i in range(1, _NUM_DF):
            slot = i % 2
            nxt = (i + 1) % 2
            if i + 1 < _NUM_DF:
                stage_idx(i + 1, nxt)
                nxt_handles = fire_gathers(nxt, rows_bufs[nxt])
            else:
                nxt_handles = []
            for h in pending:
                h.wait()
            accumulate(slot)
            pending = nxt_handles
        pltpu.sync_copy(acc, out_hbm.at[pl.ds(wid * _BPW, _BPW)])

    return body(didx2d, tables_flat)


def _tc_mlp_body(cf_ref, w1_ref, b1_ref, w2_ref, b2_ref, emb_ref, out_ref):
    x = cf_ref[...]
    x = jnp.where(jnp.isnan(x), 0.0, x)
    h = jnp.maximum(
        jnp.dot(x, w1_ref[...], preferred_element_type=jnp.float32)
        + b1_ref[...], 0.0)
    h = jnp.clip(h, -65000.0, 65000.0)
    o = jnp.maximum(
        jnp.dot(h, w2_ref[...], preferred_element_type=jnp.float32)
        + b2_ref[...], 0.0)
    out_ref[...] = o + emb_ref[...]


def _tc_mlp(cf_mat, w1t, b1, w2t, b2, embsum):
    blk = 2048
    grid = _BATCH // blk
    return pl.pallas_call(
        _tc_mlp_body,
        grid=(grid,),
        in_specs=[
            pl.BlockSpec((blk, _NUM_CF), lambda i: (i, 0)),
            pl.BlockSpec((_NUM_CF, 2 * _NUM_CF), lambda i: (0, 0)),
            pl.BlockSpec((1, 2 * _NUM_CF), lambda i: (0, 0)),
            pl.BlockSpec((2 * _NUM_CF, _EMBED), lambda i: (0, 0)),
            pl.BlockSpec((1, _EMBED), lambda i: (0, 0)),
            pl.BlockSpec((blk, _EMBED), lambda i: (i, 0)),
        ],
        out_specs=pl.BlockSpec((blk, _EMBED), lambda i: (i, 0)),
        out_shape=jax.ShapeDtypeStruct((_BATCH, _EMBED), jnp.float32),
    )(cf_mat, w1t, b1.reshape(1, -1), w2t, b2.reshape(1, -1), embsum)


def kernel(cf_0, cf_1, cf_2, cf_3, cf_4, cf_5, cf_6, cf_7,
           df_0, df_1, df_2, df_3, df_4, df_5, df_6, df_7, df_8, df_9,
           df_10, df_11, df_12, df_13, df_14, df_15, df_16, df_17, df_18,
           df_19, df_20, df_21, df_22, df_23, df_24, df_25,
           W1, b1, W2, b2, tables):
    cfs = [cf_0, cf_1, cf_2, cf_3, cf_4, cf_5, cf_6, cf_7]
    dfs = [df_0, df_1, df_2, df_3, df_4, df_5, df_6, df_7, df_8, df_9,
           df_10, df_11, df_12, df_13, df_14, df_15, df_16, df_17, df_18,
           df_19, df_20, df_21, df_22, df_23, df_24, df_25]
    cf_mat = jnp.stack(cfs, axis=1)
    didx2d = jnp.stack(dfs, axis=0).reshape(-1, _IDXW)
    tables_flat = tables.reshape(_NUM_DF * _VOCAB, _EMBED)
    embsum = _sc_embsum(didx2d, tables_flat)
    return _tc_mlp(cf_mat, W1.T, b1, W2.T, b2, embsum)


# speedup vs baseline: 1.0641x; 1.0027x over previous
import functools

import jax
import jax.numpy as jnp
from jax import lax
from jax.experimental import pallas as pl
from jax.experimental.pallas import tpu as pltpu
from jax.experimental.pallas import tpu_sc as plsc

_NUM_CF = 8
_NUM_DF = 26
_VOCAB = 100000
_EMBED = 64
_BATCH = 16384

_INFO = plsc.get_sparse_core_info()
_NC = _INFO.num_cores
_NS = _INFO.num_subcores
_NW = _NC * _NS
_BPW = _BATCH // _NW
_IDXW = 128
_NG = _BPW // _IDXW


def _sc_embsum(didx2d, tables3d):
    mesh = plsc.VectorSubcoreMesh(core_axis_name="c", subcore_axis_name="s")

    @functools.partial(
        pl.kernel,
        out_type=jax.ShapeDtypeStruct((_BATCH, _EMBED), jnp.float32),
        mesh=mesh,
        scratch_types=[
            pltpu.VMEM((_NG, _IDXW), jnp.int32),
            pltpu.VMEM((_NG, _IDXW), jnp.int32),
            pltpu.VMEM((_BPW, _EMBED), jnp.float32),
            pltpu.VMEM((_BPW, _EMBED), jnp.float32),
            pltpu.VMEM((_BPW, _EMBED), jnp.float32),
            pltpu.SemaphoreType.DMA,
            pltpu.SemaphoreType.DMA,
        ],
        compiler_params=pltpu.CompilerParams(use_tc_tiling_on_sc=False),
    )
    def body(didx_hbm, tab_hbm, out_hbm, idx0, idx1, rows0, rows1, acc, sem0, sem1):
        wid = lax.axis_index("s") * _NC + lax.axis_index("c")
        idx_bufs = (idx0, idx1)
        rows_bufs = (rows0, rows1)
        sems = (sem0, sem1)
        rows_per_feat = _BATCH // _IDXW

        def stage_idx(i, slot):
            ib = idx_bufs[slot]
            pltpu.sync_copy(
                didx_hbm.at[pl.ds(i * rows_per_feat + wid * _NG, _NG)], ib)

        def fire_gathers(i, slot, dst):
            ib = idx_bufs[slot]
            handles = []
            for g in range(_NG):
                handles.append(
                    pltpu.async_copy(
                        tab_hbm.at[i].at[ib.at[g]],
                        dst.at[pl.ds(g * _IDXW, _IDXW)],
                        sems[slot],
                    ))
            return handles

        def accumulate(slot):
            rb = rows_bufs[slot]

            def accrow(r, _):
                for c in range(_EMBED // 16):
                    plsc.addupdate(
                        acc.at[r, pl.ds(c * 16, 16)],
                        rb[r, pl.ds(c * 16, 16)])
                return 0

            lax.fori_loop(0, _BPW, accrow, 0, unroll=4)

        stage_idx(0, 0)
        h0 = fire_gathers(0, 0, acc)
        stage_idx(1, 1)
        h1 = fire_gathers(1, 1, rows_bufs[1])
        for h in h0:
            h.wait()
        pending = h1
        for i in range(1, _NUM_DF):
            slot = i % 2
            nxt = (i + 1) % 2
            if i + 1 < _NUM_DF:
                stage_idx(i + 1, nxt)
                nxt_handles = fire_gathers(i + 1, nxt, rows_bufs[nxt])
            else:
                nxt_handles = []
            for h in pending:
                h.wait()
            accumulate(slot)
            pending = nxt_handles
        pltpu.sync_copy(acc, out_hbm.at[pl.ds(wid * _BPW, _BPW)])

    return body(didx2d, tables3d)


def _tc_mlp_body(cf_ref, w1_ref, b1_ref, w2_ref, b2_ref, emb_ref, out_ref):
    x = cf_ref[...]
    x = jnp.where(jnp.isnan(x), 0.0, x)
    h = jnp.maximum(
        jnp.dot(x, w1_ref[...], preferred_element_type=jnp.float32)
        + b1_ref[...], 0.0)
    h = jnp.clip(h, -65000.0, 65000.0)
    o = jnp.maximum(
        jnp.dot(h, w2_ref[...], preferred_element_type=jnp.float32)
        + b2_ref[...], 0.0)
    out_ref[...] = o + emb_ref[...]


def _tc_mlp(cf_mat, w1t, b1, w2t, b2, embsum):
    blk = 2048
    grid = _BATCH // blk
    return pl.pallas_call(
        _tc_mlp_body,
        grid=(grid,),
        in_specs=[
            pl.BlockSpec((blk, _NUM_CF), lambda i: (i, 0)),
            pl.BlockSpec((_NUM_CF, 2 * _NUM_CF), lambda i: (0, 0)),
            pl.BlockSpec((1, 2 * _NUM_CF), lambda i: (0, 0)),
            pl.BlockSpec((2 * _NUM_CF, _EMBED), lambda i: (0, 0)),
            pl.BlockSpec((1, _EMBED), lambda i: (0, 0)),
            pl.BlockSpec((blk, _EMBED), lambda i: (i, 0)),
        ],
        out_specs=pl.BlockSpec((blk, _EMBED), lambda i: (i, 0)),
        out_shape=jax.ShapeDtypeStruct((_BATCH, _EMBED), jnp.float32),
    )(cf_mat, w1t, b1.reshape(1, -1), w2t, b2.reshape(1, -1), embsum)


def kernel(cf_0, cf_1, cf_2, cf_3, cf_4, cf_5, cf_6, cf_7,
           df_0, df_1, df_2, df_3, df_4, df_5, df_6, df_7, df_8, df_9,
           df_10, df_11, df_12, df_13, df_14, df_15, df_16, df_17, df_18,
           df_19, df_20, df_21, df_22, df_23, df_24, df_25,
           W1, b1, W2, b2, tables):
    cfs = [cf_0, cf_1, cf_2, cf_3, cf_4, cf_5, cf_6, cf_7]
    dfs = [df_0, df_1, df_2, df_3, df_4, df_5, df_6, df_7, df_8, df_9,
           df_10, df_11, df_12, df_13, df_14, df_15, df_16, df_17, df_18,
           df_19, df_20, df_21, df_22, df_23, df_24, df_25]
    cf_mat = jnp.stack(cfs, axis=1)
    didx2d = jnp.stack(dfs, axis=0).reshape(-1, _IDXW)
    embsum = _sc_embsum(didx2d, tables)
    return _tc_mlp(cf_mat, W1.T, b1, W2.T, b2, embsum)


# speedup vs baseline: 1.8027x; 1.6941x over previous
import functools

import jax
import jax.numpy as jnp
from jax import lax
from jax.experimental import pallas as pl
from jax.experimental.pallas import tpu as pltpu
from jax.experimental.pallas import tpu_sc as plsc

_NUM_CF = 8
_NUM_DF = 26
_VOCAB = 100000
_EMBED = 64
_BATCH = 16384

_INFO = plsc.get_sparse_core_info()
_NC = _INFO.num_cores
_NS = _INFO.num_subcores
_NW = _NC * _NS
_BPW = _BATCH // _NW
_IDXW = 128
_HALF = _BPW // 4
_NG = _HALF // _IDXW

_SPLIT = 49920
_PROWS = _VOCAB - _SPLIT
_CHUNK = 2048
_NFULL = _PROWS // _CHUNK
_TAIL = _PROWS - _NFULL * _CHUNK


def _xpose_body(in_ref, out_hbm, ybuf0, ybuf1, sem0, sem1):
    i = pl.program_id(0)
    ybufs = (ybuf0, ybuf1)
    sems = (sem0, sem1)
    x = in_ref

    prev = [None, None]
    for k in range(_NFULL + 1):
        b = k % 2
        a = k * _CHUNK
        n = _CHUNK if k < _NFULL else _TAIL
        if prev[b] is not None:
            prev[b].wait()
        ylo = jnp.transpose(x[0, :, a:a + n], (1, 0))
        yhi = jnp.transpose(x[0, :, _SPLIT + a:_SPLIT + a + n], (1, 0))
        ybufs[b][0:n, 0:_EMBED] = ylo
        ybufs[b][0:n, _EMBED:2 * _EMBED] = yhi
        cp = pltpu.make_async_copy(
            ybufs[b].at[pl.ds(0, n)],
            out_hbm.at[i, pl.ds(a, n)],
            sems[b])
        cp.start()
        prev[b] = cp
    for b in range(2):
        if prev[b] is not None:
            prev[b].wait()


def _tc_format_table(tables_t):
    return pl.pallas_call(
        _xpose_body,
        grid=(_NUM_DF,),
        in_specs=[pl.BlockSpec((1, _EMBED, _VOCAB), lambda i: (i, 0, 0))],
        out_specs=pl.BlockSpec(memory_space=pl.ANY),
        out_shape=jax.ShapeDtypeStruct((_NUM_DF, _PROWS, 2 * _EMBED),
                                       jnp.float32),
        scratch_shapes=[
            pltpu.VMEM((_CHUNK, 2 * _EMBED), jnp.float32),
            pltpu.VMEM((_CHUNK, 2 * _EMBED), jnp.float32),
            pltpu.SemaphoreType.DMA,
            pltpu.SemaphoreType.DMA,
        ],
        compiler_params=pltpu.CompilerParams(
            vmem_limit_bytes=60 * 1024 * 1024),
    )(tables_t)


def _sc_embsum(didx2d, tables_p):
    mesh = plsc.VectorSubcoreMesh(core_axis_name="c", subcore_axis_name="s")

    @functools.partial(
        pl.kernel,
        out_type=jax.ShapeDtypeStruct((_BATCH, _EMBED), jnp.float32),
        mesh=mesh,
        scratch_types=[
            pltpu.VMEM((_NG, _IDXW), jnp.int32),
            pltpu.VMEM((_NG, _IDXW), jnp.int32),
            pltpu.VMEM((_NG, _IDXW), jnp.int32),
            pltpu.VMEM((_NG, _IDXW), jnp.int32),
            pltpu.VMEM((_HALF, 2 * _EMBED), jnp.float32),
            pltpu.VMEM((_HALF, 2 * _EMBED), jnp.float32),
            pltpu.VMEM((_BPW, _EMBED), jnp.float32),
            pltpu.SemaphoreType.DMA,
            pltpu.SemaphoreType.DMA,
        ],
    )
    def body(didx_hbm, tab_hbm, out_hbm,
             raw0, raw1, pair0, pair1, rows0, rows1, acc, sem0, sem1):
        wid = lax.axis_index("s") * _NC + lax.axis_index("c")
        raw_bufs = (raw0, raw1)
        pair_bufs = (pair0, pair1)
        rows_bufs = (rows0, rows1)
        sems = (sem0, sem1)
        rows_per_feat = _BATCH // _IDXW
        per_feat = _BPW // _HALF
        nhalves = per_feat * _NUM_DF

        zero16 = jnp.zeros((16,), jnp.float32)

        def stage_and_fire(h, slot):
            i = h // per_feat
            sub = h % per_feat
            rawb, pairb = raw_bufs[slot], pair_bufs[slot]
            pltpu.sync_copy(
                didx_hbm.at[pl.ds(i * rows_per_feat
                                  + wid * per_feat * _NG
                                  + sub * _NG, _NG)],
                rawb)
            for g in range(_NG):
                for c in range(_IDXW // 16):
                    v = rawb[g, pl.ds(c * 16, 16)]
                    pairb[g, pl.ds(c * 16, 16)] = jnp.where(
                        v < _SPLIT, v, v - _SPLIT)
            for g in range(_NG):
                pltpu.make_async_copy(
                    tab_hbm.at[i].at[pairb.at[g]],
                    rows_bufs[slot].at[pl.ds(g * _IDXW, _IDXW)],
                    sems[slot],
                ).start()

        def wait_gathers(h, slot):
            i = h // per_feat
            pairb = pair_bufs[slot]
            for g in range(_NG):
                pltpu.make_async_copy(
                    tab_hbm.at[i].at[pairb.at[g]],
                    rows_bufs[slot].at[pl.ds(g * _IDXW, _IDXW)],
                    sems[slot],
                ).wait()

        def accumulate(h, slot):
            rb = rows_bufs[slot]
            rawb = raw_bufs[slot]
            sub = h % per_feat
            for g in range(_NG):

                def accblk(j, _, g=g):
                    v16 = rawb[g, pl.ds(j * 16, 16)]
                    for rr in range(16):
                        off = jnp.where(v16[rr] < _SPLIT, 0, _EMBED)
                        r = g * _IDXW + j * 16 + rr
                        arow = sub * _HALF + r
                        for c in range(_EMBED // 16):
                            plsc.addupdate(
                                acc.at[arow, pl.ds(c * 16, 16)],
                                rb[r, pl.ds(off + c * 16, 16)])
                    return 0

                lax.fori_loop(0, _IDXW // 16, accblk, 0)

        stage_and_fire(0, 0)

        def zrow(r, _):
            for c in range(_EMBED // 16):
                acc[r, pl.ds(c * 16, 16)] = zero16
            return 0

        lax.fori_loop(0, _BPW, zrow, 0, unroll=4)

        def loop_body(h, _):
            for slot in range(2):

                @pl.when(h % 2 == slot)
                def _(slot=slot):
                    wait_gathers(h, slot)
                    @pl.when(h + 1 < nhalves)
                    def _():
                        stage_and_fire(h + 1, 1 - slot)
                    accumulate(h, slot)

            return 0

        lax.fori_loop(0, nhalves, loop_body, 0)
        pltpu.sync_copy(acc, out_hbm.at[pl.ds(wid * _BPW, _BPW)])

    return body(didx2d, tables_p)


def _tc_mlp_body(cf_ref, w1_ref, b1_ref, w2_ref, b2_ref, emb_ref, out_ref):
    x = cf_ref[...]
    x = jnp.where(jnp.isnan(x), 0.0, x)
    h = jnp.maximum(
        jnp.dot(x, w1_ref[...], preferred_element_type=jnp.float32)
        + b1_ref[...], 0.0)
    h = jnp.clip(h, -65000.0, 65000.0)
    o = jnp.maximum(
        jnp.dot(h, w2_ref[...], preferred_element_type=jnp.float32)
        + b2_ref[...], 0.0)
    out_ref[...] = o + emb_ref[...]


def _tc_mlp(cf_mat, w1t, b1, w2t, b2, embsum):
    blk = 2048
    grid = _BATCH // blk
    return pl.pallas_call(
        _tc_mlp_body,
        grid=(grid,),
        in_specs=[
            pl.BlockSpec((blk, _NUM_CF), lambda i: (i, 0)),
            pl.BlockSpec((_NUM_CF, 2 * _NUM_CF), lambda i: (0, 0)),
            pl.BlockSpec((1, 2 * _NUM_CF), lambda i: (0, 0)),
            pl.BlockSpec((2 * _NUM_CF, _EMBED), lambda i: (0, 0)),
            pl.BlockSpec((1, _EMBED), lambda i: (0, 0)),
            pl.BlockSpec((blk, _EMBED), lambda i: (i, 0)),
        ],
        out_specs=pl.BlockSpec((blk, _EMBED), lambda i: (i, 0)),
        out_shape=jax.ShapeDtypeStruct((_BATCH, _EMBED), jnp.float32),
    )(cf_mat, w1t, b1.reshape(1, -1), w2t, b2.reshape(1, -1), embsum)


def kernel(cf_0, cf_1, cf_2, cf_3, cf_4, cf_5, cf_6, cf_7,
           df_0, df_1, df_2, df_3, df_4, df_5, df_6, df_7, df_8, df_9,
           df_10, df_11, df_12, df_13, df_14, df_15, df_16, df_17, df_18,
           df_19, df_20, df_21, df_22, df_23, df_24, df_25,
           W1, b1, W2, b2, tables):
    cfs = [cf_0, cf_1, cf_2, cf_3, cf_4, cf_5, cf_6, cf_7]
    dfs = [df_0, df_1, df_2, df_3, df_4, df_5, df_6, df_7, df_8, df_9,
           df_10, df_11, df_12, df_13, df_14, df_15, df_16, df_17, df_18,
           df_19, df_20, df_21, df_22, df_23, df_24, df_25]
    cf_mat = jnp.stack(cfs, axis=1)
    didx2d = jnp.stack(dfs, axis=0).reshape(-1, _IDXW)
    tables_t = jnp.transpose(tables, (0, 2, 1))
    tables_p = _tc_format_table(tables_t)
    embsum = _sc_embsum(didx2d, tables_p)
    return _tc_mlp(cf_mat, W1.T, b1, W2.T, b2, embsum)
